# baseline (speedup 1.0000x reference)
import functools
import jax
import jax.numpy as jnp
from jax.experimental import pallas as pl
from jax.experimental.pallas import tpu as pltpu

B, N, E = 4, 10000, 160000
D_IN, D, D_OUT = 128, 128, 128
H = 4
HD = D // H
NUM_TYPES = 6
L = 2

BN = 1000


def _inproj_body(x_ref, nt_ref, w_ref, b_ref, te_ref, o_ref):
    x = x_ref[0]
    nt = nt_ref[0, 0]
    h = jax.lax.dot_general(x, w_ref[...], (((1,), (0,)), ((), ())),
                            precision=jax.lax.Precision.HIGHEST)
    oh = (nt[:, None] == jax.lax.broadcasted_iota(jnp.int32, (1, NUM_TYPES), 1)).astype(jnp.float32)
    h = h + jax.lax.dot_general(oh, te_ref[...], (((1,), (0,)), ((), ())),
                                precision=jax.lax.Precision.HIGHEST)
    o_ref[0] = h + b_ref[...]


def _inproj(x, nt, w, b, te):
    nb = N // BN
    x4 = x.reshape(B * nb, BN, D_IN)
    nt4 = nt.reshape(B * nb, 1, BN)
    out = pl.pallas_call(
        _inproj_body,
        grid=(B * nb,),
        in_specs=[
            pl.BlockSpec((1, BN, D_IN), lambda i: (i, 0, 0)),
            pl.BlockSpec((1, 1, BN), lambda i: (i, 0, 0)),
            pl.BlockSpec((D_IN, D), lambda i: (0, 0)),
            pl.BlockSpec((D,), lambda i: (0,)),
            pl.BlockSpec((NUM_TYPES, D), lambda i: (0, 0)),
        ],
        out_specs=pl.BlockSpec((1, BN, D), lambda i: (i, 0, 0)),
        out_shape=jax.ShapeDtypeStruct((B * nb, BN, D), jnp.float32),
    )(x4, nt4, w, b, te)
    return out.reshape(B, N, D)


def _gat_single(h, edge_index, mask, W, b, a_s, a_d):
    src = edge_index[0]
    dst = edge_index[1]
    hW = (h @ W + b).reshape(N, H, HD)
    al_s = (hW * a_s[None, :, :]).sum(-1)
    al_d = (hW * a_d[None, :, :]).sum(-1)
    e = jax.nn.leaky_relu(al_s[src] + al_d[dst], 0.2)
    m = jax.ops.segment_max(e, dst, num_segments=N)
    m = jnp.where(jnp.isfinite(m), m, 0.0)
    ex = jnp.exp(e - m[dst])
    denom = jax.ops.segment_sum(ex, dst, num_segments=N)
    attn = ex / (denom[dst] + 1e-9)
    msg = hW[src] * attn[:, :, None]
    agg = jax.ops.segment_sum(msg, dst, num_segments=N).reshape(N, D)
    out = jax.nn.elu(agg) + h
    out = out * mask[:, None]
    return out


def kernel(node_features, edge_index, node_types, node_mask, type_embed, in_W, in_b, gat_W, gat_b, a_src, a_dst, out_W, out_b):
    h = _inproj(node_features, node_types, in_W, in_b, type_embed)
    for i in range(L):
        h = jax.vmap(lambda hh, ei, mm: _gat_single(hh, ei, mm, gat_W[i], gat_b[i], a_src[i], a_dst[i]))(h, edge_index, node_mask)
    node_emb = h @ out_W + out_b
    masked = node_emb * node_mask[..., None]
    graph_emb = masked.sum(axis=1) / jnp.clip(node_mask.sum(axis=1, keepdims=True), 1.0)
    return node_emb, graph_emb


# speedup vs baseline: 78.8831x; 78.8831x over previous
import functools
import jax
import jax.numpy as jnp
from jax import lax
from jax.experimental import pallas as pl
from jax.experimental.pallas import tpu as pltpu
from jax.experimental.pallas import tpu_sc as plsc

B, N, E = 4, 10000, 160000
D_IN, D, D_OUT = 128, 128, 128
H = 4
HD = D // H
NUM_TYPES = 6
L = 2

BN = 1000
NB = N // BN

CH = 128
NSUP = 5
SUP = 16
NCH = NSUP * SUP
EPT = CH * NCH
EPAD = EPT * 16
NP = 10016
STRIPE = NP // 16
DR = 160
DC = 128
ALW = NP * 4


def _dotT(a, b):
    return lax.dot_general(a, b, (((1,), (0,)), ((), ())),
                           precision=lax.Precision.HIGHEST)


def _inproj_body(x_ref, nt_ref, w_ref, b_ref, te_ref, o_ref):
    x = x_ref[0]
    nt = nt_ref[0, 0]
    h = _dotT(x, w_ref[...])
    oh = (nt[:, None] == lax.broadcasted_iota(jnp.int32, (1, NUM_TYPES), 1)
          ).astype(jnp.float32)
    h = h + _dotT(oh, te_ref[...])
    o_ref[0] = h + b_ref[...]


def _inproj(x, nt, w, b, te):
    x4 = x.reshape(B * NB, BN, D_IN)
    nt4 = nt.reshape(B * NB, 1, BN)
    out = pl.pallas_call(
        _inproj_body,
        grid=(B * NB,),
        in_specs=[
            pl.BlockSpec((1, BN, D_IN), lambda i: (i, 0, 0)),
            pl.BlockSpec((1, 1, BN), lambda i: (i, 0, 0)),
            pl.BlockSpec((D_IN, D), lambda i: (0, 0)),
            pl.BlockSpec((D,), lambda i: (0,)),
            pl.BlockSpec((NUM_TYPES, D), lambda i: (0, 0)),
        ],
        out_specs=pl.BlockSpec((1, BN, D), lambda i: (i, 0, 0)),
        out_shape=jax.ShapeDtypeStruct((B * NB, BN, D), jnp.float32),
    )(x4, nt4, w, b, te)
    return out.reshape(B, N, D)


def _pre_body(h_ref, w_ref, b_ref, a_ref, hw_ref, al_ref):
    h = h_ref[0]
    hw = _dotT(h, w_ref[...]) + b_ref[...]
    hw_ref[0] = hw
    al_ref[0] = _dotT(hw, a_ref[...])


def _pre_layer(h, w, b, a):
    h4 = h.reshape(B * NB, BN, D)
    hw, al = pl.pallas_call(
        _pre_body,
        grid=(B * NB,),
        in_specs=[
            pl.BlockSpec((1, BN, D), lambda i: (i, 0, 0)),
            pl.BlockSpec((D, D), lambda i: (0, 0)),
            pl.BlockSpec((D,), lambda i: (0,)),
            pl.BlockSpec((D, 8), lambda i: (0, 0)),
        ],
        out_specs=[
            pl.BlockSpec((1, BN, D), lambda i: (i, 0, 0)),
            pl.BlockSpec((1, BN, 8), lambda i: (i, 0, 0)),
        ],
        out_shape=[
            jax.ShapeDtypeStruct((B * NB, BN, D), jnp.float32),
            jax.ShapeDtypeStruct((B * NB, BN, 8), jnp.float32),
        ],
    )(h4, w, b, a)
    return hw.reshape(B, N, D), al.reshape(B, N, 8)


def _post_body(h_ref, unn_ref, den_ref, s_ref, m_ref, o_ref):
    h = h_ref[0]
    unn = unn_ref[0]
    rep = _dotT(den_ref[0], s_ref[...])
    agg = unn / (rep + 1e-9)
    out = jnp.where(agg > 0.0, agg, jnp.exp(agg) - 1.0) + h
    o_ref[0] = out * m_ref[0, 0][:, None]


def _post_layer(h, unn, den, sel, mask):
    h4 = h.reshape(B * NB, BN, D)
    unn4 = unn.reshape(B * NB, BN, D)
    den4 = den.reshape(B * NB, BN, H)
    m4 = mask.reshape(B * NB, 1, BN)
    out = pl.pallas_call(
        _post_body,
        grid=(B * NB,),
        in_specs=[
            pl.BlockSpec((1, BN, D), lambda i: (i, 0, 0)),
            pl.BlockSpec((1, BN, D), lambda i: (i, 0, 0)),
            pl.BlockSpec((1, BN, H), lambda i: (i, 0, 0)),
            pl.BlockSpec((H, D), lambda i: (0, 0)),
            pl.BlockSpec((1, 1, BN), lambda i: (i, 0, 0)),
        ],
        out_specs=pl.BlockSpec((1, BN, D), lambda i: (i, 0, 0)),
        out_shape=jax.ShapeDtypeStruct((B * NB, BN, D), jnp.float32),
    )(h4, unn4, den4, sel, m4)
    return out.reshape(B, N, D)


def _final_body(h_ref, w_ref, b_ref, m_ref, ne_ref, sums_ref):
    i = pl.program_id(1)
    h = h_ref[0, 0]
    ne = _dotT(h, w_ref[...]) + b_ref[...]
    ne_ref[0, 0] = ne
    mvec = m_ref[0, 0, 0]
    masked = ne * mvec[:, None]
    part = jnp.sum(masked, axis=0, keepdims=True)
    msum = jnp.sum(mvec)
    prow = jnp.concatenate([part, jnp.full((1, D), msum, jnp.float32)], axis=0)

    @pl.when(i == 0)
    def _():
        sums_ref[0] = prow

    @pl.when(i > 0)
    def _():
        sums_ref[0] = sums_ref[0] + prow


def _final(h, w, b, mask):
    h4 = h.reshape(B, NB, BN, D)
    m4 = mask.reshape(B, NB, 1, BN)
    ne, sums = pl.pallas_call(
        _final_body,
        grid=(B, NB),
        in_specs=[
            pl.BlockSpec((1, 1, BN, D), lambda bb, i: (bb, i, 0, 0)),
            pl.BlockSpec((D, D_OUT), lambda bb, i: (0, 0)),
            pl.BlockSpec((D_OUT,), lambda bb, i: (0,)),
            pl.BlockSpec((1, 1, 1, BN), lambda bb, i: (bb, i, 0, 0)),
        ],
        out_specs=[
            pl.BlockSpec((1, 1, BN, D_OUT), lambda bb, i: (bb, i, 0, 0)),
            pl.BlockSpec((1, 2, D_OUT), lambda bb, i: (bb, 0, 0)),
        ],
        out_shape=[
            jax.ShapeDtypeStruct((B, NB, BN, D_OUT), jnp.float32),
            jax.ShapeDtypeStruct((B, 2, D_OUT), jnp.float32),
        ],
    )(h4, w, b, m4)
    node_emb = ne.reshape(B, N, D_OUT)
    graph_emb = sums[:, 0, :] / jnp.clip(sums[:, 1, :], 1.0, None)
    return node_emb, graph_emb


def _sc_body(edge_hbm, al_hbm, hw_hbm, unn_out, den_out,
             alv, src_all, dst_all, denom_loc, rows, ex_buf, zflat,
             idx_a, idx_b, unnorm_sh, denom_sh, sem_g):
    c = lax.axis_index("c")
    tid = lax.axis_index("s")
    r0 = tid * STRIPE
    _F32Z = jnp.zeros((16,), jnp.float32)

    lanes = lax.iota(jnp.int32, 16)

    def fill_a(g, u):
        idx_a[pl.ds(g * 16, 16)] = lanes + g * 16
        return u
    lax.fori_loop(0, 8, fill_a, 0)

    def fill_b(g, u):
        idx_b[pl.ds(g * 16, 16)] = lanes + (128 + g * 16)
        return u
    lax.fori_loop(0, 2, fill_b, 0)

    def graph_body(b, carry):
        pltpu.sync_copy(al_hbm.at[b, c], alv)

        def z_rows(i, u):
            for q in range(4):
                rows[i, pl.ds(q * 16, 16)] = _F32Z
            return u
        lax.fori_loop(0, CH, z_rows, 0)

        def z_flat(i, u):
            for q in range(8):
                zflat[i, pl.ds(q * 16, 16)] = _F32Z
            return u
        lax.fori_loop(0, DR // 16, z_flat, 0)

        def z_den(i, u):
            for q in range(8):
                denom_loc[i, pl.ds(q * 16, 16)] = _F32Z
            return u
        lax.fori_loop(0, DR, z_den, 0)

        for k in range(4):
            pltpu.sync_copy(rows, unnorm_sh.at[pl.ds(r0 + k * 128, 128), :])
        pltpu.sync_copy(rows.at[pl.ds(0, STRIPE - 512)],
                        unnorm_sh.at[pl.ds(r0 + 512, STRIPE - 512), :])
        pltpu.sync_copy(zflat, denom_sh.at[pl.ds(tid * (DR // 16), DR // 16), :])
        plsc.subcore_barrier()

        def superblock(s, u):
            pltpu.sync_copy(edge_hbm.at[b, 0, tid, s], src_all)
            pltpu.sync_copy(edge_hbm.at[b, 1, tid, s], dst_all)
            lax.fori_loop(0, SUP, chunk, 0)
            return u

        def chunk(j, u):
            gather = pltpu.async_copy(
                hw_hbm.at[b, c].at[src_all.at[j]], rows, sem_g)
            for g in range(CH // 16):
                s16 = src_all[j, pl.ds(g * 16, 16)]
                d16 = dst_all[j, pl.ds(g * 16, 16)]
                s4 = s16 * 4
                d4 = d16 * 4
                for h in range(2):
                    sval = plsc.load_gather(alv, [s4 + h])
                    dval = plsc.load_gather(alv, [d4 + (2 + h)])
                    e = sval + dval
                    e = jnp.where(e >= 0.0, e, e * jnp.float32(0.2))
                    exv = jnp.exp(e)
                    ex_buf[pl.ds(g * 32 + h * 16, 16)] = exv
                    fi = d16 * 2 + h
                    plsc.addupdate_scatter(
                        denom_loc, [fi >> 7, fi & 127], exv)
            gather.wait()
            for g in range(CH // 16):
                for h in range(2):
                    exv = ex_buf[pl.ds(g * 32 + h * 16, 16)]
                    for l in range(16):
                        ee = g * 16 + l
                        scv = jnp.full((16,), exv[l], jnp.float32)
                        for q in range(2):
                            sl = pl.ds(h * 32 + q * 16, 16)
                            rows[ee, sl] = rows[ee, sl] * scv
            pltpu.sync_copy(rows, unnorm_sh.at[dst_all.at[j]], add=True)
            return u
        lax.fori_loop(0, NSUP, superblock, 0)

        pltpu.sync_copy(denom_loc.at[pl.ds(0, 128)],
                        denom_sh.at[idx_a], add=True)
        pltpu.sync_copy(denom_loc.at[pl.ds(128, 32)],
                        denom_sh.at[idx_b], add=True)
        plsc.subcore_barrier()
        for k in range(5):
            rc = 128 if k < 4 else STRIPE - 512
            pltpu.sync_copy(unnorm_sh.at[pl.ds(r0 + k * 128, rc), :],
                            rows.at[pl.ds(0, rc)])
            pltpu.sync_copy(rows.at[pl.ds(0, rc)],
                            unn_out.at[b, c, pl.ds(r0 + k * 128, rc), :])
        pltpu.sync_copy(denom_sh.at[pl.ds(tid * (DR // 16), DR // 16), :],
                        zflat)
        pltpu.sync_copy(zflat, den_out.at[b, c, tid])
        plsc.subcore_barrier()
        return carry

    lax.fori_loop(0, B, graph_body, 0)


@functools.partial(
    pl.kernel,
    mesh=plsc.VectorSubcoreMesh(core_axis_name="c", subcore_axis_name="s"),
    compiler_params=pltpu.CompilerParams(use_tc_tiling_on_sc=False,
                                         needs_layout_passes=False),
    out_type=[
        jax.ShapeDtypeStruct((B, 2, NP, 64), jnp.float32),
        jax.ShapeDtypeStruct((B, 2, 16, DR // 16, DC), jnp.float32),
    ],
    scratch_types=[
        pltpu.VMEM((ALW,), jnp.float32),
        pltpu.VMEM((SUP, CH), jnp.int32),
        pltpu.VMEM((SUP, CH), jnp.int32),
        pltpu.VMEM((DR, DC), jnp.float32),
        pltpu.VMEM((CH, 64), jnp.float32),
        pltpu.VMEM((256,), jnp.float32),
        pltpu.VMEM((DR // 16, DC), jnp.float32),
        pltpu.VMEM((128,), jnp.int32),
        pltpu.VMEM((32,), jnp.int32),
        pltpu.VMEM_SHARED((NP, 64), jnp.float32),
        pltpu.VMEM_SHARED((DR, DC), jnp.float32),
        pltpu.SemaphoreType.DMA,
    ],
)
def _sc_edge_kernel(edge_hbm, al_hbm, hw_hbm, unn_out, den_out,
                    alv, src_all, dst_all, denom_loc, rows, ex_buf, zflat,
                    idx_a, idx_b, unnorm_sh, denom_sh, sem_g):
    _sc_body(edge_hbm, al_hbm, hw_hbm, unn_out, den_out,
             alv, src_all, dst_all, denom_loc, rows, ex_buf, zflat,
             idx_a, idx_b, unnorm_sh, denom_sh, sem_g)


def _layer(h, edge_sc, mask, w, b, a_sel, sel):
    hw, al = _pre_layer(h, w, b, a_sel)

    hw_p = jnp.pad(hw, ((0, 0), (0, NP - N), (0, 0)))
    hw_sc = hw_p.reshape(B, NP, 2, 64).transpose(0, 2, 1, 3)

    al_r = jnp.stack([al[:, :, (0, 1, 4, 5)], al[:, :, (2, 3, 6, 7)]], axis=1)
    al_p = jnp.pad(al_r, ((0, 0), (0, 0), (0, NP - N), (0, 0)))
    al_sc = al_p.reshape(B, 2, ALW)

    unn_sc, den_sc = _sc_edge_kernel(edge_sc, al_sc, hw_sc)

    unn = unn_sc[:, :, :N, :].transpose(0, 2, 1, 3).reshape(B, N, D)
    den = den_sc.reshape(B, 2, DR * DC // 2, 2)[:, :, :N, :]
    den = den.transpose(0, 2, 1, 3).reshape(B, N, H)

    return _post_layer(h, unn, den, sel, mask)


def kernel(node_features, edge_index, node_types, node_mask, type_embed,
           in_W, in_b, gat_W, gat_b, a_src, a_dst, out_W, out_b):
    f32 = jnp.float32

    eyeh = jnp.repeat(jnp.eye(H, dtype=f32), HD, axis=0)
    a_s_mat = eyeh * a_src.reshape(L, 1, D).transpose(0, 2, 1)
    a_d_mat = eyeh * a_dst.reshape(L, 1, D).transpose(0, 2, 1)
    a_sel = jnp.concatenate([a_s_mat, a_d_mat], axis=-1)

    sel = jnp.repeat(jnp.eye(H, dtype=f32), HD, axis=1)

    ei = edge_index.astype(jnp.int32)
    ei_pad = jnp.concatenate(
        [ei, jnp.full((B, 2, EPAD - E), N, jnp.int32)], axis=2)
    edge_sc = ei_pad.reshape(B, 2, 16, NSUP, SUP, CH)

    h = _inproj(node_features, node_types, in_W, in_b, type_embed)
    for i in range(L):
        h = _layer(h, edge_sc, node_mask, gat_W[i], gat_b[i],
                   a_sel[i], sel)
    return _final(h, out_W, out_b, node_mask)


# speedup vs baseline: 82.5616x; 1.0466x over previous
import functools
import jax
import jax.numpy as jnp
from jax import lax
from jax.experimental import pallas as pl
from jax.experimental.pallas import tpu as pltpu
from jax.experimental.pallas import tpu_sc as plsc

B, N, E = 4, 10000, 160000
D_IN, D, D_OUT = 128, 128, 128
H = 4
HD = D // H
NUM_TYPES = 6
L = 2

BN = 1000
NB = N // BN

CH = 128
NSUP = 5
SUP = 16
NCH = NSUP * SUP
EPT = CH * NCH
EPAD = EPT * 16
NP = 10016
STRIPE = NP // 16
DR = 160
DC = 128
ALW = NP * 4


def _dotT(a, b):
    return lax.dot_general(a, b, (((1,), (0,)), ((), ())),
                           precision=lax.Precision.HIGHEST)


def _inproj_body(x_ref, nt_ref, w_ref, b_ref, te_ref, o_ref):
    x = x_ref[0]
    nt = nt_ref[0, 0]
    h = _dotT(x, w_ref[...])
    oh = (nt[:, None] == lax.broadcasted_iota(jnp.int32, (1, NUM_TYPES), 1)
          ).astype(jnp.float32)
    h = h + _dotT(oh, te_ref[...])
    o_ref[0] = h + b_ref[...]


def _inproj(x, nt, w, b, te):
    x4 = x.reshape(B * NB, BN, D_IN)
    nt4 = nt.reshape(B * NB, 1, BN)
    out = pl.pallas_call(
        _inproj_body,
        grid=(B * NB,),
        in_specs=[
            pl.BlockSpec((1, BN, D_IN), lambda i: (i, 0, 0)),
            pl.BlockSpec((1, 1, BN), lambda i: (i, 0, 0)),
            pl.BlockSpec((D_IN, D), lambda i: (0, 0)),
            pl.BlockSpec((D,), lambda i: (0,)),
            pl.BlockSpec((NUM_TYPES, D), lambda i: (0, 0)),
        ],
        out_specs=pl.BlockSpec((1, BN, D), lambda i: (i, 0, 0)),
        out_shape=jax.ShapeDtypeStruct((B * NB, BN, D), jnp.float32),
    )(x4, nt4, w, b, te)
    return out.reshape(B, N, D)


def _pre_body(h_ref, w_ref, b_ref, a_ref, hw_ref, al_ref):
    h = h_ref[0]
    hw = _dotT(h, w_ref[...]) + b_ref[...]
    hw_ref[0] = hw
    al_ref[0] = _dotT(hw, a_ref[...])


def _pre_layer(h, w, b, a):
    h4 = h.reshape(B * NB, BN, D)
    hw, al = pl.pallas_call(
        _pre_body,
        grid=(B * NB,),
        in_specs=[
            pl.BlockSpec((1, BN, D), lambda i: (i, 0, 0)),
            pl.BlockSpec((D, D), lambda i: (0, 0)),
            pl.BlockSpec((D,), lambda i: (0,)),
            pl.BlockSpec((D, 8), lambda i: (0, 0)),
        ],
        out_specs=[
            pl.BlockSpec((1, BN, D), lambda i: (i, 0, 0)),
            pl.BlockSpec((1, BN, 8), lambda i: (i, 0, 0)),
        ],
        out_shape=[
            jax.ShapeDtypeStruct((B * NB, BN, D), jnp.float32),
            jax.ShapeDtypeStruct((B * NB, BN, 8), jnp.float32),
        ],
    )(h4, w, b, a)
    return hw.reshape(B, N, D), al.reshape(B, N, 8)


def _post_body(h_ref, unn_ref, den_ref, s_ref, m_ref, o_ref):
    h = h_ref[0]
    unn = unn_ref[0]
    rep = _dotT(den_ref[0], s_ref[...])
    agg = unn / (rep + 1e-9)
    out = jnp.where(agg > 0.0, agg, jnp.exp(agg) - 1.0) + h
    o_ref[0] = out * m_ref[0, 0][:, None]


def _post_layer(h, unn, den, sel, mask):
    h4 = h.reshape(B * NB, BN, D)
    unn4 = unn.reshape(B * NB, BN, D)
    den4 = den.reshape(B * NB, BN, H)
    m4 = mask.reshape(B * NB, 1, BN)
    out = pl.pallas_call(
        _post_body,
        grid=(B * NB,),
        in_specs=[
            pl.BlockSpec((1, BN, D), lambda i: (i, 0, 0)),
            pl.BlockSpec((1, BN, D), lambda i: (i, 0, 0)),
            pl.BlockSpec((1, BN, H), lambda i: (i, 0, 0)),
            pl.BlockSpec((H, D), lambda i: (0, 0)),
            pl.BlockSpec((1, 1, BN), lambda i: (i, 0, 0)),
        ],
        out_specs=pl.BlockSpec((1, BN, D), lambda i: (i, 0, 0)),
        out_shape=jax.ShapeDtypeStruct((B * NB, BN, D), jnp.float32),
    )(h4, unn4, den4, sel, m4)
    return out.reshape(B, N, D)


def _final_body(h_ref, w_ref, b_ref, m_ref, ne_ref, sums_ref):
    i = pl.program_id(1)
    h = h_ref[0, 0]
    ne = _dotT(h, w_ref[...]) + b_ref[...]
    ne_ref[0, 0] = ne
    mvec = m_ref[0, 0, 0]
    masked = ne * mvec[:, None]
    part = jnp.sum(masked, axis=0, keepdims=True)
    msum = jnp.sum(mvec)
    prow = jnp.concatenate([part, jnp.full((1, D), msum, jnp.float32)], axis=0)

    @pl.when(i == 0)
    def _():
        sums_ref[0] = prow

    @pl.when(i > 0)
    def _():
        sums_ref[0] = sums_ref[0] + prow


def _final(h, w, b, mask):
    h4 = h.reshape(B, NB, BN, D)
    m4 = mask.reshape(B, NB, 1, BN)
    ne, sums = pl.pallas_call(
        _final_body,
        grid=(B, NB),
        in_specs=[
            pl.BlockSpec((1, 1, BN, D), lambda bb, i: (bb, i, 0, 0)),
            pl.BlockSpec((D, D_OUT), lambda bb, i: (0, 0)),
            pl.BlockSpec((D_OUT,), lambda bb, i: (0,)),
            pl.BlockSpec((1, 1, 1, BN), lambda bb, i: (bb, i, 0, 0)),
        ],
        out_specs=[
            pl.BlockSpec((1, 1, BN, D_OUT), lambda bb, i: (bb, i, 0, 0)),
            pl.BlockSpec((1, 2, D_OUT), lambda bb, i: (bb, 0, 0)),
        ],
        out_shape=[
            jax.ShapeDtypeStruct((B, NB, BN, D_OUT), jnp.float32),
            jax.ShapeDtypeStruct((B, 2, D_OUT), jnp.float32),
        ],
    )(h4, w, b, m4)
    node_emb = ne.reshape(B, N, D_OUT)
    graph_emb = sums[:, 0, :] / jnp.clip(sums[:, 1, :], 1.0, None)
    return node_emb, graph_emb


def _sc_body(edge_hbm, al_hbm, hw_hbm, unn_out, den_out,
             alv, src_all, dst_all, denom_loc, rows, rows1,
             ex_buf, ex_buf1, zflat, idx_a, idx_b,
             unnorm_sh, denom_sh, sem_g, sem_g1, sem_s, sem_s1):
    c = lax.axis_index("c")
    tid = lax.axis_index("s")
    r0 = tid * STRIPE
    _F32Z = jnp.zeros((16,), jnp.float32)

    lanes = lax.iota(jnp.int32, 16)

    def fill_a(g, u):
        idx_a[pl.ds(g * 16, 16)] = lanes + g * 16
        return u
    lax.fori_loop(0, 8, fill_a, 0)

    def fill_b(g, u):
        idx_b[pl.ds(g * 16, 16)] = lanes + (128 + g * 16)
        return u
    lax.fori_loop(0, 2, fill_b, 0)

    def graph_body(b, carry):
        pltpu.sync_copy(al_hbm.at[b, c], alv)

        def z_rows(i, u):
            for q in range(4):
                rows[i, pl.ds(q * 16, 16)] = _F32Z
            return u
        lax.fori_loop(0, CH, z_rows, 0)

        def z_flat(i, u):
            for q in range(8):
                zflat[i, pl.ds(q * 16, 16)] = _F32Z
            return u
        lax.fori_loop(0, DR // 16, z_flat, 0)

        def z_den(i, u):
            for q in range(8):
                denom_loc[i, pl.ds(q * 16, 16)] = _F32Z
            return u
        lax.fori_loop(0, DR, z_den, 0)

        for k in range(4):
            pltpu.sync_copy(rows, unnorm_sh.at[pl.ds(r0 + k * 128, 128), :])
        pltpu.sync_copy(rows.at[pl.ds(0, STRIPE - 512)],
                        unnorm_sh.at[pl.ds(r0 + 512, STRIPE - 512), :])
        pltpu.sync_copy(zflat, denom_sh.at[pl.ds(tid * (DR // 16), DR // 16), :])
        plsc.subcore_barrier()

        def compute_ex(j, exb):
            for g in range(CH // 16):
                s16 = src_all[j, pl.ds(g * 16, 16)]
                d16 = dst_all[j, pl.ds(g * 16, 16)]
                s4 = s16 * 4
                d4 = d16 * 4
                for h in range(2):
                    sval = plsc.load_gather(alv, [s4 + h])
                    dval = plsc.load_gather(alv, [d4 + (2 + h)])
                    e = sval + dval
                    e = jnp.where(e >= 0.0, e, e * jnp.float32(0.2))
                    exv = jnp.exp(e)
                    exb[pl.ds(g * 32 + h * 16, 16)] = exv
                    fi = d16 * 2 + h
                    plsc.addupdate_scatter(
                        denom_loc, [fi >> 7, fi & 127], exv)

        def scale_rows(buf, exb):
            for g in range(CH // 16):
                for h in range(2):
                    exv = exb[pl.ds(g * 32 + h * 16, 16)]
                    for l in range(16):
                        ee = g * 16 + l
                        scv = jnp.full((16,), exv[l], jnp.float32)
                        for q in range(2):
                            sl = pl.ds(h * 32 + q * 16, 16)
                            buf[ee, sl] = buf[ee, sl] * scv

        def pair(jj, u):
            j0 = jj * 2
            j1 = j0 + 1
            g0 = pltpu.async_copy(
                hw_hbm.at[b, c].at[src_all.at[j0]], rows, sem_g)
            g1 = pltpu.async_copy(
                hw_hbm.at[b, c].at[src_all.at[j1]], rows1, sem_g1)
            compute_ex(j0, ex_buf)
            compute_ex(j1, ex_buf1)
            g0.wait()
            scale_rows(rows, ex_buf)
            s0 = pltpu.async_copy(rows, unnorm_sh.at[dst_all.at[j0]],
                                  sem_s, add=True)
            g1.wait()
            scale_rows(rows1, ex_buf1)
            s1 = pltpu.async_copy(rows1, unnorm_sh.at[dst_all.at[j1]],
                                  sem_s1, add=True)
            s0.wait()
            s1.wait()
            return u

        def superblock(s, u):
            pltpu.sync_copy(edge_hbm.at[b, 0, tid, s], src_all)
            pltpu.sync_copy(edge_hbm.at[b, 1, tid, s], dst_all)
            lax.fori_loop(0, SUP // 2, pair, 0)
            return u

        lax.fori_loop(0, NSUP, superblock, 0)

        pltpu.sync_copy(denom_loc.at[pl.ds(0, 128)],
                        denom_sh.at[idx_a], add=True)
        pltpu.sync_copy(denom_loc.at[pl.ds(128, 32)],
                        denom_sh.at[idx_b], add=True)
        plsc.subcore_barrier()
        for k in range(5):
            rc = 128 if k < 4 else STRIPE - 512
            pltpu.sync_copy(unnorm_sh.at[pl.ds(r0 + k * 128, rc), :],
                            rows.at[pl.ds(0, rc)])
            pltpu.sync_copy(rows.at[pl.ds(0, rc)],
                            unn_out.at[b, c, pl.ds(r0 + k * 128, rc), :])
        pltpu.sync_copy(denom_sh.at[pl.ds(tid * (DR // 16), DR // 16), :],
                        zflat)
        pltpu.sync_copy(zflat, den_out.at[b, c, tid])
        plsc.subcore_barrier()
        return carry

    lax.fori_loop(0, B, graph_body, 0)


@functools.partial(
    pl.kernel,
    mesh=plsc.VectorSubcoreMesh(core_axis_name="c", subcore_axis_name="s"),
    compiler_params=pltpu.CompilerParams(use_tc_tiling_on_sc=False,
                                         needs_layout_passes=False),
    out_type=[
        jax.ShapeDtypeStruct((B, 2, NP, 64), jnp.float32),
        jax.ShapeDtypeStruct((B, 2, 16, DR // 16, DC), jnp.float32),
    ],
    scratch_types=[
        pltpu.VMEM((ALW,), jnp.float32),
        pltpu.VMEM((SUP, CH), jnp.int32),
        pltpu.VMEM((SUP, CH), jnp.int32),
        pltpu.VMEM((DR, DC), jnp.float32),
        pltpu.VMEM((CH, 64), jnp.float32),
        pltpu.VMEM((CH, 64), jnp.float32),
        pltpu.VMEM((256,), jnp.float32),
        pltpu.VMEM((256,), jnp.float32),
        pltpu.VMEM((DR // 16, DC), jnp.float32),
        pltpu.VMEM((128,), jnp.int32),
        pltpu.VMEM((32,), jnp.int32),
        pltpu.VMEM_SHARED((NP, 64), jnp.float32),
        pltpu.VMEM_SHARED((DR, DC), jnp.float32),
        pltpu.SemaphoreType.DMA,
        pltpu.SemaphoreType.DMA,
        pltpu.SemaphoreType.DMA,
        pltpu.SemaphoreType.DMA,
    ],
)
def _sc_edge_kernel(edge_hbm, al_hbm, hw_hbm, unn_out, den_out,
                    alv, src_all, dst_all, denom_loc, rows, rows1,
                    ex_buf, ex_buf1, zflat, idx_a, idx_b,
                    unnorm_sh, denom_sh, sem_g, sem_g1, sem_s, sem_s1):
    _sc_body(edge_hbm, al_hbm, hw_hbm, unn_out, den_out,
             alv, src_all, dst_all, denom_loc, rows, rows1,
             ex_buf, ex_buf1, zflat, idx_a, idx_b,
             unnorm_sh, denom_sh, sem_g, sem_g1, sem_s, sem_s1)


def _layer(h, edge_sc, mask, w, b, a_sel, sel):
    hw, al = _pre_layer(h, w, b, a_sel)

    hw_p = jnp.pad(hw, ((0, 0), (0, NP - N), (0, 0)))
    hw_sc = hw_p.reshape(B, NP, 2, 64).transpose(0, 2, 1, 3)

    al_r = jnp.stack([al[:, :, (0, 1, 4, 5)], al[:, :, (2, 3, 6, 7)]], axis=1)
    al_p = jnp.pad(al_r, ((0, 0), (0, 0), (0, NP - N), (0, 0)))
    al_sc = al_p.reshape(B, 2, ALW)

    unn_sc, den_sc = _sc_edge_kernel(edge_sc, al_sc, hw_sc)

    unn = unn_sc[:, :, :N, :].transpose(0, 2, 1, 3).reshape(B, N, D)
    den = den_sc.reshape(B, 2, DR * DC // 2, 2)[:, :, :N, :]
    den = den.transpose(0, 2, 1, 3).reshape(B, N, H)

    return _post_layer(h, unn, den, sel, mask)


def kernel(node_features, edge_index, node_types, node_mask, type_embed,
           in_W, in_b, gat_W, gat_b, a_src, a_dst, out_W, out_b):
    f32 = jnp.float32

    eyeh = jnp.repeat(jnp.eye(H, dtype=f32), HD, axis=0)
    a_s_mat = eyeh * a_src.reshape(L, 1, D).transpose(0, 2, 1)
    a_d_mat = eyeh * a_dst.reshape(L, 1, D).transpose(0, 2, 1)
    a_sel = jnp.concatenate([a_s_mat, a_d_mat], axis=-1)

    sel = jnp.repeat(jnp.eye(H, dtype=f32), HD, axis=1)

    ei = edge_index.astype(jnp.int32)
    ei_pad = jnp.concatenate(
        [ei, jnp.full((B, 2, EPAD - E), N, jnp.int32)], axis=2)
    edge_sc = ei_pad.reshape(B, 2, 16, NSUP, SUP, CH)

    h = _inproj(node_features, node_types, in_W, in_b, type_embed)
    for i in range(L):
        h = _layer(h, edge_sc, node_mask, gat_W[i], gat_b[i],
                   a_sel[i], sel)
    return _final(h, out_W, out_b, node_mask)


# speedup vs baseline: 99.3964x; 1.2039x over previous
import functools
import jax
import jax.numpy as jnp
from jax import lax
from jax.experimental import pallas as pl
from jax.experimental.pallas import tpu as pltpu
from jax.experimental.pallas import tpu_sc as plsc

B, N, E = 4, 10000, 160000
D_IN, D, D_OUT = 128, 128, 128
H = 4
HD = D // H
NUM_TYPES = 6
L = 2

BN = 1000
NB = N // BN

CH = 128
NSUP = 5
SUP = 16
NCH = NSUP * SUP
EPT = CH * NCH
EPAD = EPT * 16
NP = 10016
STRIPE = NP // 16
DR = 160
DC = 128
ALW = NP * 4


def _dotT(a, b):
    return lax.dot_general(a, b, (((1,), (0,)), ((), ())),
                           precision=lax.Precision.HIGHEST)


def _inproj_body(x_ref, nt_ref, w_ref, b_ref, te_ref, o_ref):
    x = x_ref[0]
    nt = nt_ref[0, 0]
    h = _dotT(x, w_ref[...])
    oh = (nt[:, None] == lax.broadcasted_iota(jnp.int32, (1, NUM_TYPES), 1)
          ).astype(jnp.float32)
    h = h + _dotT(oh, te_ref[...])
    o_ref[0] = h + b_ref[...]


def _inproj(x, nt, w, b, te):
    x4 = x.reshape(B * NB, BN, D_IN)
    nt4 = nt.reshape(B * NB, 1, BN)
    out = pl.pallas_call(
        _inproj_body,
        grid=(B * NB,),
        in_specs=[
            pl.BlockSpec((1, BN, D_IN), lambda i: (i, 0, 0)),
            pl.BlockSpec((1, 1, BN), lambda i: (i, 0, 0)),
            pl.BlockSpec((D_IN, D), lambda i: (0, 0)),
            pl.BlockSpec((D,), lambda i: (0,)),
            pl.BlockSpec((NUM_TYPES, D), lambda i: (0, 0)),
        ],
        out_specs=pl.BlockSpec((1, BN, D), lambda i: (i, 0, 0)),
        out_shape=jax.ShapeDtypeStruct((B * NB, BN, D), jnp.float32),
    )(x4, nt4, w, b, te)
    return out.reshape(B, N, D)


def _pre_body(h_ref, w_ref, b_ref, a_ref, hw_ref, al_ref):
    h = h_ref[0]
    hw = _dotT(h, w_ref[...]) + b_ref[...]
    hw_ref[0, 0] = hw[:, :64]
    hw_ref[0, 1] = hw[:, 64:]
    al8 = _dotT(hw, a_ref[...])
    al_ref[0, 0] = al8[:, :4]
    al_ref[0, 1] = al8[:, 4:]


def _pre_layer(h, w, b, a):
    h4 = h.reshape(B * NB, BN, D)
    hw_sc, al_sc = pl.pallas_call(
        _pre_body,
        grid=(B * NB,),
        in_specs=[
            pl.BlockSpec((1, BN, D), lambda i: (i, 0, 0)),
            pl.BlockSpec((D, D), lambda i: (0, 0)),
            pl.BlockSpec((D,), lambda i: (0,)),
            pl.BlockSpec((D, 8), lambda i: (0, 0)),
        ],
        out_specs=[
            pl.BlockSpec((1, 2, BN, 64), lambda i: (i // NB, 0, i % NB, 0)),
            pl.BlockSpec((1, 2, BN, 4), lambda i: (i // NB, 0, i % NB, 0)),
        ],
        out_shape=[
            jax.ShapeDtypeStruct((B, 2, NP, 64), jnp.float32),
            jax.ShapeDtypeStruct((B, 2, NP, 4), jnp.float32),
        ],
    )(h4, w, b, a)
    return hw_sc, al_sc


def _post_body(h_ref, u0_ref, u1_ref, den_ref, s_ref, m_ref, o_ref):
    h = h_ref[0]
    unn = jnp.concatenate([u0_ref[0, 0], u1_ref[0, 0]], axis=-1)
    rep = _dotT(den_ref[0], s_ref[...])
    agg = unn / (rep + 1e-9)
    out = jnp.where(agg > 0.0, agg, jnp.exp(agg) - 1.0) + h
    o_ref[0] = out * m_ref[0, 0][:, None]


def _post_layer(h, unn_sc, den, sel, mask):
    h4 = h.reshape(B * NB, BN, D)
    den4 = den.reshape(B * NB, BN, H)
    m4 = mask.reshape(B * NB, 1, BN)
    out = pl.pallas_call(
        _post_body,
        grid=(B * NB,),
        in_specs=[
            pl.BlockSpec((1, BN, D), lambda i: (i, 0, 0)),
            pl.BlockSpec((1, 1, BN, 64), lambda i: (i // NB, 0, i % NB, 0)),
            pl.BlockSpec((1, 1, BN, 64), lambda i: (i // NB, 1, i % NB, 0)),
            pl.BlockSpec((1, BN, H), lambda i: (i, 0, 0)),
            pl.BlockSpec((H, D), lambda i: (0, 0)),
            pl.BlockSpec((1, 1, BN), lambda i: (i, 0, 0)),
        ],
        out_specs=pl.BlockSpec((1, BN, D), lambda i: (i, 0, 0)),
        out_shape=jax.ShapeDtypeStruct((B * NB, BN, D), jnp.float32),
    )(h4, unn_sc, unn_sc, den4, sel, m4)
    return out.reshape(B, N, D)


def _final_body(h_ref, w_ref, b_ref, m_ref, ne_ref, sums_ref):
    i = pl.program_id(1)
    h = h_ref[0, 0]
    ne = _dotT(h, w_ref[...]) + b_ref[...]
    ne_ref[0, 0] = ne
    mvec = m_ref[0, 0, 0]
    masked = ne * mvec[:, None]
    part = jnp.sum(masked, axis=0, keepdims=True)
    msum = jnp.sum(mvec)
    prow = jnp.concatenate([part, jnp.full((1, D), msum, jnp.float32)], axis=0)

    @pl.when(i == 0)
    def _():
        sums_ref[0] = prow

    @pl.when(i > 0)
    def _():
        sums_ref[0] = sums_ref[0] + prow


def _final(h, w, b, mask):
    h4 = h.reshape(B, NB, BN, D)
    m4 = mask.reshape(B, NB, 1, BN)
    ne, sums = pl.pallas_call(
        _final_body,
        grid=(B, NB),
        in_specs=[
            pl.BlockSpec((1, 1, BN, D), lambda bb, i: (bb, i, 0, 0)),
            pl.BlockSpec((D, D_OUT), lambda bb, i: (0, 0)),
            pl.BlockSpec((D_OUT,), lambda bb, i: (0,)),
            pl.BlockSpec((1, 1, 1, BN), lambda bb, i: (bb, i, 0, 0)),
        ],
        out_specs=[
            pl.BlockSpec((1, 1, BN, D_OUT), lambda bb, i: (bb, i, 0, 0)),
            pl.BlockSpec((1, 2, D_OUT), lambda bb, i: (bb, 0, 0)),
        ],
        out_shape=[
            jax.ShapeDtypeStruct((B, NB, BN, D_OUT), jnp.float32),
            jax.ShapeDtypeStruct((B, 2, D_OUT), jnp.float32),
        ],
    )(h4, w, b, m4)
    node_emb = ne.reshape(B, N, D_OUT)
    graph_emb = sums[:, 0, :] / jnp.clip(sums[:, 1, :], 1.0, None)
    return node_emb, graph_emb


def _sc_body(edge_hbm, al_hbm, hw_hbm, unn_out, den_out,
             alv, src_all, dst_all, denom_loc, rows, rows1,
             ex_buf, ex_buf1, zflat, idx_a, idx_b,
             unnorm_sh, denom_sh, sem_g, sem_g1, sem_s, sem_s1):
    c = lax.axis_index("c")
    tid = lax.axis_index("s")
    r0 = tid * STRIPE
    _F32Z = jnp.zeros((16,), jnp.float32)

    lanes = lax.iota(jnp.int32, 16)

    def fill_a(g, u):
        idx_a[pl.ds(g * 16, 16)] = lanes + g * 16
        return u
    lax.fori_loop(0, 8, fill_a, 0)

    def fill_b(g, u):
        idx_b[pl.ds(g * 16, 16)] = lanes + (128 + g * 16)
        return u
    lax.fori_loop(0, 2, fill_b, 0)

    def graph_body(b, carry):
        pltpu.sync_copy(al_hbm.at[b, c], alv)

        def z_rows(i, u):
            for q in range(4):
                rows[i, pl.ds(q * 16, 16)] = _F32Z
            return u
        lax.fori_loop(0, CH, z_rows, 0)

        def z_flat(i, u):
            for q in range(8):
                zflat[i, pl.ds(q * 16, 16)] = _F32Z
            return u
        lax.fori_loop(0, DR // 16, z_flat, 0)

        def z_den(i, u):
            for q in range(8):
                denom_loc[i, pl.ds(q * 16, 16)] = _F32Z
            return u
        lax.fori_loop(0, DR, z_den, 0)

        for k in range(4):
            pltpu.sync_copy(rows, unnorm_sh.at[pl.ds(r0 + k * 128, 128), :])
        pltpu.sync_copy(rows.at[pl.ds(0, STRIPE - 512)],
                        unnorm_sh.at[pl.ds(r0 + 512, STRIPE - 512), :])
        pltpu.sync_copy(zflat, denom_sh.at[pl.ds(tid * (DR // 16), DR // 16), :])
        plsc.subcore_barrier()

        def compute_ex(j, exb):
            for g in range(CH // 16):
                s16 = src_all[j, pl.ds(g * 16, 16)]
                d16 = dst_all[j, pl.ds(g * 16, 16)]
                s4 = s16 * 4
                d4 = d16 * 4
                for h in range(2):
                    sval = plsc.load_gather(alv, [s4 + h])
                    dval = plsc.load_gather(alv, [d4 + (2 + h)])
                    e = sval + dval
                    e = jnp.where(e >= 0.0, e, e * jnp.float32(0.2))
                    exv = jnp.exp(e)
                    exb[pl.ds(g * 32 + h * 16, 16)] = exv
                    fi = d16 * 2 + h
                    plsc.addupdate_scatter(
                        denom_loc, [fi >> 7, fi & 127], exv)

        def scale_rows(buf, exb):
            for g in range(CH // 16):
                for h in range(2):
                    exv = exb[pl.ds(g * 32 + h * 16, 16)]
                    for l in range(16):
                        ee = g * 16 + l
                        scv = jnp.full((16,), exv[l], jnp.float32)
                        for q in range(2):
                            sl = pl.ds(h * 32 + q * 16, 16)
                            buf[ee, sl] = buf[ee, sl] * scv

        def pair(jj, u):
            j0 = jj * 2
            j1 = j0 + 1
            g0 = pltpu.async_copy(
                hw_hbm.at[b, c].at[src_all.at[j0]], rows, sem_g)
            g1 = pltpu.async_copy(
                hw_hbm.at[b, c].at[src_all.at[j1]], rows1, sem_g1)
            compute_ex(j0, ex_buf)
            compute_ex(j1, ex_buf1)
            g0.wait()
            scale_rows(rows, ex_buf)
            s0 = pltpu.async_copy(rows, unnorm_sh.at[dst_all.at[j0]],
                                  sem_s, add=True)
            g1.wait()
            scale_rows(rows1, ex_buf1)
            s1 = pltpu.async_copy(rows1, unnorm_sh.at[dst_all.at[j1]],
                                  sem_s1, add=True)
            s0.wait()
            s1.wait()
            return u

        def superblock(s, u):
            pltpu.sync_copy(edge_hbm.at[b, 0, tid, s], src_all)
            pltpu.sync_copy(edge_hbm.at[b, 1, tid, s], dst_all)
            lax.fori_loop(0, SUP // 2, pair, 0)
            return u

        lax.fori_loop(0, NSUP, superblock, 0)

        pltpu.sync_copy(denom_loc.at[pl.ds(0, 128)],
                        denom_sh.at[idx_a], add=True)
        pltpu.sync_copy(denom_loc.at[pl.ds(128, 32)],
                        denom_sh.at[idx_b], add=True)
        plsc.subcore_barrier()
        for k in range(5):
            rc = 128 if k < 4 else STRIPE - 512
            pltpu.sync_copy(unnorm_sh.at[pl.ds(r0 + k * 128, rc), :],
                            rows.at[pl.ds(0, rc)])
            pltpu.sync_copy(rows.at[pl.ds(0, rc)],
                            unn_out.at[b, c, pl.ds(r0 + k * 128, rc), :])
        pltpu.sync_copy(denom_sh.at[pl.ds(tid * (DR // 16), DR // 16), :],
                        zflat)
        pltpu.sync_copy(zflat, den_out.at[b, c, tid])
        plsc.subcore_barrier()
        return carry

    lax.fori_loop(0, B, graph_body, 0)


@functools.partial(
    pl.kernel,
    mesh=plsc.VectorSubcoreMesh(core_axis_name="c", subcore_axis_name="s"),
    compiler_params=pltpu.CompilerParams(use_tc_tiling_on_sc=False,
                                         needs_layout_passes=False),
    out_type=[
        jax.ShapeDtypeStruct((B, 2, NP, 64), jnp.float32),
        jax.ShapeDtypeStruct((B, 2, 16, DR // 16, DC), jnp.float32),
    ],
    scratch_types=[
        pltpu.VMEM((ALW,), jnp.float32),
        pltpu.VMEM((SUP, CH), jnp.int32),
        pltpu.VMEM((SUP, CH), jnp.int32),
        pltpu.VMEM((DR, DC), jnp.float32),
        pltpu.VMEM((CH, 64), jnp.float32),
        pltpu.VMEM((CH, 64), jnp.float32),
        pltpu.VMEM((256,), jnp.float32),
        pltpu.VMEM((256,), jnp.float32),
        pltpu.VMEM((DR // 16, DC), jnp.float32),
        pltpu.VMEM((128,), jnp.int32),
        pltpu.VMEM((32,), jnp.int32),
        pltpu.VMEM_SHARED((NP, 64), jnp.float32),
        pltpu.VMEM_SHARED((DR, DC), jnp.float32),
        pltpu.SemaphoreType.DMA,
        pltpu.SemaphoreType.DMA,
        pltpu.SemaphoreType.DMA,
        pltpu.SemaphoreType.DMA,
    ],
)
def _sc_edge_kernel(edge_hbm, al_hbm, hw_hbm, unn_out, den_out,
                    alv, src_all, dst_all, denom_loc, rows, rows1,
                    ex_buf, ex_buf1, zflat, idx_a, idx_b,
                    unnorm_sh, denom_sh, sem_g, sem_g1, sem_s, sem_s1):
    _sc_body(edge_hbm, al_hbm, hw_hbm, unn_out, den_out,
             alv, src_all, dst_all, denom_loc, rows, rows1,
             ex_buf, ex_buf1, zflat, idx_a, idx_b,
             unnorm_sh, denom_sh, sem_g, sem_g1, sem_s, sem_s1)


def _layer(h, edge_sc, mask, w, b, a_sel, sel):
    hw_sc, al_sc = _pre_layer(h, w, b, a_sel)

    unn_sc, den_sc = _sc_edge_kernel(edge_sc, al_sc.reshape(B, 2, ALW), hw_sc)

    den = den_sc.reshape(B, 2, DR * DC // 2, 2)[:, :, :N, :]
    den = den.transpose(0, 2, 1, 3).reshape(B, N, H)

    return _post_layer(h, unn_sc, den, sel, mask)


def kernel(node_features, edge_index, node_types, node_mask, type_embed,
           in_W, in_b, gat_W, gat_b, a_src, a_dst, out_W, out_b):
    f32 = jnp.float32

    eyeh = jnp.repeat(jnp.eye(H, dtype=f32), HD, axis=0)
    a_s_mat = eyeh * a_src.reshape(L, 1, D).transpose(0, 2, 1)
    a_d_mat = eyeh * a_dst.reshape(L, 1, D).transpose(0, 2, 1)
    a_sel = jnp.concatenate([a_s_mat, a_d_mat], axis=-1)
    a_sel = a_sel[:, :, jnp.array([0, 1, 4, 5, 2, 3, 6, 7])]

    sel = jnp.repeat(jnp.eye(H, dtype=f32), HD, axis=1)

    ei = edge_index.astype(jnp.int32)
    ei_pad = jnp.concatenate(
        [ei, jnp.full((B, 2, EPAD - E), N, jnp.int32)], axis=2)
    edge_sc = ei_pad.reshape(B, 2, 16, NSUP, SUP, CH)

    h = _inproj(node_features, node_types, in_W, in_b, type_embed)
    for i in range(L):
        h = _layer(h, edge_sc, node_mask, gat_W[i], gat_b[i],
                   a_sel[i], sel)
    return _final(h, out_W, out_b, node_mask)


# speedup vs baseline: 101.3219x; 1.0194x over previous
import functools
import jax
import jax.numpy as jnp
from jax import lax
from jax.experimental import pallas as pl
from jax.experimental.pallas import tpu as pltpu
from jax.experimental.pallas import tpu_sc as plsc

B, N, E = 4, 10000, 160000
D_IN, D, D_OUT = 128, 128, 128
H = 4
HD = D // H
NUM_TYPES = 6
L = 2

BN = 1000
NB = N // BN

CH = 128
NSUP = 5
SUP = 16
NCH = NSUP * SUP
EPT = CH * NCH
EPAD = EPT * 16
NP = 10016
STRIPE = NP // 16
DR = 160
DC = 128
ALW = NP * 4


def _dotT(a, b):
    return lax.dot_general(a, b, (((1,), (0,)), ((), ())),
                           precision=lax.Precision.HIGHEST)


def _inproj_body(x_ref, nt_ref, w_ref, b_ref, te_ref, o_ref):
    x = x_ref[0]
    nt = nt_ref[0, 0]
    h = _dotT(x, w_ref[...])
    oh = (nt[:, None] == lax.broadcasted_iota(jnp.int32, (1, NUM_TYPES), 1)
          ).astype(jnp.float32)
    h = h + _dotT(oh, te_ref[...])
    o_ref[0] = h + b_ref[...]


def _inproj(x, nt, w, b, te):
    x4 = x.reshape(B * NB, BN, D_IN)
    nt4 = nt.reshape(B * NB, 1, BN)
    out = pl.pallas_call(
        _inproj_body,
        grid=(B * NB,),
        in_specs=[
            pl.BlockSpec((1, BN, D_IN), lambda i: (i, 0, 0)),
            pl.BlockSpec((1, 1, BN), lambda i: (i, 0, 0)),
            pl.BlockSpec((D_IN, D), lambda i: (0, 0)),
            pl.BlockSpec((D,), lambda i: (0,)),
            pl.BlockSpec((NUM_TYPES, D), lambda i: (0, 0)),
        ],
        out_specs=pl.BlockSpec((1, BN, D), lambda i: (i, 0, 0)),
        out_shape=jax.ShapeDtypeStruct((B * NB, BN, D), jnp.float32),
    )(x4, nt4, w, b, te)
    return out.reshape(B, N, D)


def _pre_body(h_ref, w_ref, b_ref, a_ref, hw_ref, al_ref):
    h = h_ref[0]
    hw = _dotT(h, w_ref[...]) + b_ref[...]
    hw_ref[0, 0] = hw[:, :64]
    hw_ref[0, 1] = hw[:, 64:]
    al8 = _dotT(hw, a_ref[...])
    al_ref[0, 0] = al8[:, :4]
    al_ref[0, 1] = al8[:, 4:]


def _pre_layer(h, w, b, a):
    h4 = h.reshape(B * NB, BN, D)
    hw_sc, al_sc = pl.pallas_call(
        _pre_body,
        grid=(B * NB,),
        in_specs=[
            pl.BlockSpec((1, BN, D), lambda i: (i, 0, 0)),
            pl.BlockSpec((D, D), lambda i: (0, 0)),
            pl.BlockSpec((D,), lambda i: (0,)),
            pl.BlockSpec((D, 8), lambda i: (0, 0)),
        ],
        out_specs=[
            pl.BlockSpec((1, 2, BN, 64), lambda i: (i // NB, 0, i % NB, 0)),
            pl.BlockSpec((1, 2, BN, 4), lambda i: (i // NB, 0, i % NB, 0)),
        ],
        out_shape=[
            jax.ShapeDtypeStruct((B, 2, NP, 64), jnp.float32),
            jax.ShapeDtypeStruct((B, 2, NP, 4), jnp.float32),
        ],
    )(h4, w, b, a)
    return hw_sc, al_sc


def _post_body(h_ref, u0_ref, u1_ref, den_ref, s_ref, m_ref, o_ref):
    h = h_ref[0]
    unn = jnp.concatenate([u0_ref[0, 0], u1_ref[0, 0]], axis=-1)
    rep = _dotT(den_ref[0], s_ref[...])
    agg = unn / (rep + 1e-9)
    out = jnp.where(agg > 0.0, agg, jnp.exp(agg) - 1.0) + h
    o_ref[0] = out * m_ref[0, 0][:, None]


def _post_layer(h, unn_sc, den, sel, mask):
    h4 = h.reshape(B * NB, BN, D)
    den4 = den.reshape(B * NB, BN, H)
    m4 = mask.reshape(B * NB, 1, BN)
    out = pl.pallas_call(
        _post_body,
        grid=(B * NB,),
        in_specs=[
            pl.BlockSpec((1, BN, D), lambda i: (i, 0, 0)),
            pl.BlockSpec((1, 1, BN, 64), lambda i: (i // NB, 0, i % NB, 0)),
            pl.BlockSpec((1, 1, BN, 64), lambda i: (i // NB, 1, i % NB, 0)),
            pl.BlockSpec((1, BN, H), lambda i: (i, 0, 0)),
            pl.BlockSpec((H, D), lambda i: (0, 0)),
            pl.BlockSpec((1, 1, BN), lambda i: (i, 0, 0)),
        ],
        out_specs=pl.BlockSpec((1, BN, D), lambda i: (i, 0, 0)),
        out_shape=jax.ShapeDtypeStruct((B * NB, BN, D), jnp.float32),
    )(h4, unn_sc, unn_sc, den4, sel, m4)
    return out.reshape(B, N, D)


def _final_body(h_ref, w_ref, b_ref, m_ref, ne_ref, sums_ref):
    i = pl.program_id(1)
    h = h_ref[0, 0]
    ne = _dotT(h, w_ref[...]) + b_ref[...]
    ne_ref[0, 0] = ne
    mvec = m_ref[0, 0, 0]
    masked = ne * mvec[:, None]
    part = jnp.sum(masked, axis=0, keepdims=True)
    msum = jnp.sum(mvec)
    prow = jnp.concatenate([part, jnp.full((1, D), msum, jnp.float32)], axis=0)

    @pl.when(i == 0)
    def _():
        sums_ref[0] = prow

    @pl.when(i > 0)
    def _():
        sums_ref[0] = sums_ref[0] + prow


def _final(h, w, b, mask):
    h4 = h.reshape(B, NB, BN, D)
    m4 = mask.reshape(B, NB, 1, BN)
    ne, sums = pl.pallas_call(
        _final_body,
        grid=(B, NB),
        in_specs=[
            pl.BlockSpec((1, 1, BN, D), lambda bb, i: (bb, i, 0, 0)),
            pl.BlockSpec((D, D_OUT), lambda bb, i: (0, 0)),
            pl.BlockSpec((D_OUT,), lambda bb, i: (0,)),
            pl.BlockSpec((1, 1, 1, BN), lambda bb, i: (bb, i, 0, 0)),
        ],
        out_specs=[
            pl.BlockSpec((1, 1, BN, D_OUT), lambda bb, i: (bb, i, 0, 0)),
            pl.BlockSpec((1, 2, D_OUT), lambda bb, i: (bb, 0, 0)),
        ],
        out_shape=[
            jax.ShapeDtypeStruct((B, NB, BN, D_OUT), jnp.float32),
            jax.ShapeDtypeStruct((B, 2, D_OUT), jnp.float32),
        ],
    )(h4, w, b, m4)
    node_emb = ne.reshape(B, N, D_OUT)
    graph_emb = sums[:, 0, :] / jnp.clip(sums[:, 1, :], 1.0, None)
    return node_emb, graph_emb


def _sc_body(edge_hbm, al_hbm, hw_hbm, unn_out, den_out,
             alv, src_all, dst_all, denom_loc, rows, rows1,
             ex_buf, ex_buf1, zflat, idx_a, idx_b,
             unnorm_sh, denom_sh, sem_g, sem_g1, sem_s, sem_s1):
    c = lax.axis_index("c")
    tid = lax.axis_index("s")
    r0 = tid * STRIPE
    _F32Z = jnp.zeros((16,), jnp.float32)

    lanes = lax.iota(jnp.int32, 16)

    def fill_a(g, u):
        idx_a[pl.ds(g * 16, 16)] = lanes + g * 16
        return u
    lax.fori_loop(0, 8, fill_a, 0)

    def fill_b(g, u):
        idx_b[pl.ds(g * 16, 16)] = lanes + (128 + g * 16)
        return u
    lax.fori_loop(0, 2, fill_b, 0)

    def graph_body(b, carry):
        al_d = pltpu.async_copy(al_hbm.at[b, c], alv, sem_g)

        def z_rows(i, u):
            for q in range(4):
                rows[i, pl.ds(q * 16, 16)] = _F32Z
            return u
        lax.fori_loop(0, CH, z_rows, 0)

        def z_flat(i, u):
            for q in range(8):
                zflat[i, pl.ds(q * 16, 16)] = _F32Z
            return u
        lax.fori_loop(0, DR // 16, z_flat, 0)

        def z_den(i, u):
            for q in range(8):
                denom_loc[i, pl.ds(q * 16, 16)] = _F32Z
            return u
        lax.fori_loop(0, DR, z_den, 0)

        zds = []
        for k in range(4):
            zds.append(pltpu.async_copy(
                rows, unnorm_sh.at[pl.ds(r0 + k * 128, 128), :], sem_s))
        zds.append(pltpu.async_copy(
            rows.at[pl.ds(0, STRIPE - 512)],
            unnorm_sh.at[pl.ds(r0 + 512, STRIPE - 512), :], sem_s))
        zds.append(pltpu.async_copy(
            zflat, denom_sh.at[pl.ds(tid * (DR // 16), DR // 16), :], sem_s1))
        al_d.wait()
        for d in zds:
            d.wait()
        plsc.subcore_barrier()

        def compute_ex(j, exb):
            for g in range(CH // 16):
                s16 = src_all[j, pl.ds(g * 16, 16)]
                d16 = dst_all[j, pl.ds(g * 16, 16)]
                s4 = s16 * 4
                d4 = d16 * 4
                for h in range(2):
                    sval = plsc.load_gather(alv, [s4 + h])
                    dval = plsc.load_gather(alv, [d4 + (2 + h)])
                    e = sval + dval
                    e = jnp.where(e >= 0.0, e, e * jnp.float32(0.2))
                    exv = jnp.exp(e)
                    exb[pl.ds(g * 32 + h * 16, 16)] = exv
                    fi = d16 * 2 + h
                    plsc.addupdate_scatter(
                        denom_loc, [fi >> 7, fi & 127], exv)

        def scale_rows(buf, exb):
            for g in range(CH // 16):
                for h in range(2):
                    exv = exb[pl.ds(g * 32 + h * 16, 16)]
                    for l in range(16):
                        ee = g * 16 + l
                        scv = jnp.full((16,), exv[l], jnp.float32)
                        for q in range(2):
                            sl = pl.ds(h * 32 + q * 16, 16)
                            buf[ee, sl] = buf[ee, sl] * scv

        def pair(jj, u):
            j0 = jj * 2
            j1 = j0 + 1
            g0 = pltpu.async_copy(
                hw_hbm.at[b, c].at[src_all.at[j0]], rows, sem_g)
            g1 = pltpu.async_copy(
                hw_hbm.at[b, c].at[src_all.at[j1]], rows1, sem_g1)
            compute_ex(j0, ex_buf)
            compute_ex(j1, ex_buf1)
            g0.wait()
            scale_rows(rows, ex_buf)
            s0 = pltpu.async_copy(rows, unnorm_sh.at[dst_all.at[j0]],
                                  sem_s, add=True)
            g1.wait()
            scale_rows(rows1, ex_buf1)
            s1 = pltpu.async_copy(rows1, unnorm_sh.at[dst_all.at[j1]],
                                  sem_s1, add=True)
            s0.wait()
            s1.wait()
            return u

        def superblock(s, u):
            e0 = pltpu.async_copy(edge_hbm.at[b, 0, tid, s], src_all, sem_g)
            e1 = pltpu.async_copy(edge_hbm.at[b, 1, tid, s], dst_all, sem_g1)
            e0.wait()
            e1.wait()
            lax.fori_loop(0, SUP // 2, pair, 0)
            return u

        lax.fori_loop(0, NSUP, superblock, 0)

        pltpu.sync_copy(denom_loc.at[pl.ds(0, 128)],
                        denom_sh.at[idx_a], add=True)
        pltpu.sync_copy(denom_loc.at[pl.ds(128, 32)],
                        denom_sh.at[idx_b], add=True)
        plsc.subcore_barrier()
        pltpu.sync_copy(unnorm_sh.at[pl.ds(r0, STRIPE), :],
                        unn_out.at[b, c, pl.ds(r0, STRIPE), :])
        pltpu.sync_copy(denom_sh.at[pl.ds(tid * (DR // 16), DR // 16), :],
                        den_out.at[b, c, tid])
        plsc.subcore_barrier()
        return carry

    lax.fori_loop(0, B, graph_body, 0)


@functools.partial(
    pl.kernel,
    mesh=plsc.VectorSubcoreMesh(core_axis_name="c", subcore_axis_name="s"),
    compiler_params=pltpu.CompilerParams(use_tc_tiling_on_sc=False,
                                         needs_layout_passes=False),
    out_type=[
        jax.ShapeDtypeStruct((B, 2, NP, 64), jnp.float32),
        jax.ShapeDtypeStruct((B, 2, 16, DR // 16, DC), jnp.float32),
    ],
    scratch_types=[
        pltpu.VMEM((ALW,), jnp.float32),
        pltpu.VMEM((SUP, CH), jnp.int32),
        pltpu.VMEM((SUP, CH), jnp.int32),
        pltpu.VMEM((DR, DC), jnp.float32),
        pltpu.VMEM((CH, 64), jnp.float32),
        pltpu.VMEM((CH, 64), jnp.float32),
        pltpu.VMEM((256,), jnp.float32),
        pltpu.VMEM((256,), jnp.float32),
        pltpu.VMEM((DR // 16, DC), jnp.float32),
        pltpu.VMEM((128,), jnp.int32),
        pltpu.VMEM((32,), jnp.int32),
        pltpu.VMEM_SHARED((NP, 64), jnp.float32),
        pltpu.VMEM_SHARED((DR, DC), jnp.float32),
        pltpu.SemaphoreType.DMA,
        pltpu.SemaphoreType.DMA,
        pltpu.SemaphoreType.DMA,
        pltpu.SemaphoreType.DMA,
    ],
)
def _sc_edge_kernel(edge_hbm, al_hbm, hw_hbm, unn_out, den_out,
                    alv, src_all, dst_all, denom_loc, rows, rows1,
                    ex_buf, ex_buf1, zflat, idx_a, idx_b,
                    unnorm_sh, denom_sh, sem_g, sem_g1, sem_s, sem_s1):
    _sc_body(edge_hbm, al_hbm, hw_hbm, unn_out, den_out,
             alv, src_all, dst_all, denom_loc, rows, rows1,
             ex_buf, ex_buf1, zflat, idx_a, idx_b,
             unnorm_sh, denom_sh, sem_g, sem_g1, sem_s, sem_s1)


def _layer(h, edge_sc, mask, w, b, a_sel, sel):
    hw_sc, al_sc = _pre_layer(h, w, b, a_sel)

    unn_sc, den_sc = _sc_edge_kernel(edge_sc, al_sc.reshape(B, 2, ALW), hw_sc)

    den = den_sc.reshape(B, 2, DR * DC // 2, 2)[:, :, :N, :]
    den = den.transpose(0, 2, 1, 3).reshape(B, N, H)

    return _post_layer(h, unn_sc, den, sel, mask)


def kernel(node_features, edge_index, node_types, node_mask, type_embed,
           in_W, in_b, gat_W, gat_b, a_src, a_dst, out_W, out_b):
    f32 = jnp.float32

    eyeh = jnp.repeat(jnp.eye(H, dtype=f32), HD, axis=0)
    a_s_mat = eyeh * a_src.reshape(L, 1, D).transpose(0, 2, 1)
    a_d_mat = eyeh * a_dst.reshape(L, 1, D).transpose(0, 2, 1)
    a_sel = jnp.concatenate([a_s_mat, a_d_mat], axis=-1)
    a_sel = a_sel[:, :, jnp.array([0, 1, 4, 5, 2, 3, 6, 7])]

    sel = jnp.repeat(jnp.eye(H, dtype=f32), HD, axis=1)

    ei = edge_index.astype(jnp.int32)
    ei_pad = jnp.concatenate(
        [ei, jnp.full((B, 2, EPAD - E), N, jnp.int32)], axis=2)
    edge_sc = ei_pad.reshape(B, 2, 16, NSUP, SUP, CH)

    h = _inproj(node_features, node_types, in_W, in_b, type_embed)
    for i in range(L):
        h = _layer(h, edge_sc, node_mask, gat_W[i], gat_b[i],
                   a_sel[i], sel)
    return _final(h, out_W, out_b, node_mask)


# speedup vs baseline: 101.5174x; 1.0019x over previous
import functools
import jax
import jax.numpy as jnp
from jax import lax
from jax.experimental import pallas as pl
from jax.experimental.pallas import tpu as pltpu
from jax.experimental.pallas import tpu_sc as plsc

B, N, E = 4, 10000, 160000
D_IN, D, D_OUT = 128, 128, 128
H = 4
HD = D // H
NUM_TYPES = 6
L = 2

BN = 1000
NB = N // BN

CH = 128
NSUP = 5
SUP = 16
NCH = NSUP * SUP
EPT = CH * NCH
EPAD = EPT * 16
NP = 10016
STRIPE = NP // 16
DR = 160
DC = 128
ALW = NP * 4


def _dotT(a, b):
    return lax.dot_general(a, b, (((1,), (0,)), ((), ())),
                           precision=lax.Precision.HIGHEST)


def _inproj_body(x_ref, nt_ref, w_ref, b_ref, te_ref, o_ref):
    x = x_ref[0]
    nt = nt_ref[0, 0]
    h = _dotT(x, w_ref[...])
    oh = (nt[:, None] == lax.broadcasted_iota(jnp.int32, (1, NUM_TYPES), 1)
          ).astype(jnp.float32)
    h = h + _dotT(oh, te_ref[...])
    o_ref[0] = h + b_ref[...]


def _inproj(x, nt, w, b, te):
    x4 = x.reshape(B * NB, BN, D_IN)
    nt4 = nt.reshape(B * NB, 1, BN)
    out = pl.pallas_call(
        _inproj_body,
        grid=(B * NB,),
        in_specs=[
            pl.BlockSpec((1, BN, D_IN), lambda i: (i, 0, 0)),
            pl.BlockSpec((1, 1, BN), lambda i: (i, 0, 0)),
            pl.BlockSpec((D_IN, D), lambda i: (0, 0)),
            pl.BlockSpec((D,), lambda i: (0,)),
            pl.BlockSpec((NUM_TYPES, D), lambda i: (0, 0)),
        ],
        out_specs=pl.BlockSpec((1, BN, D), lambda i: (i, 0, 0)),
        out_shape=jax.ShapeDtypeStruct((B * NB, BN, D), jnp.float32),
    )(x4, nt4, w, b, te)
    return out.reshape(B, N, D)


def _pre_body(h_ref, w_ref, b_ref, a_ref, hw_ref, al_ref):
    h = h_ref[0]
    hw = _dotT(h, w_ref[...]) + b_ref[...]
    hw_ref[0, 0] = hw[:, :64]
    hw_ref[0, 1] = hw[:, 64:]
    al8 = _dotT(hw, a_ref[...])
    al_ref[0, 0] = al8[:, :4]
    al_ref[0, 1] = al8[:, 4:]


def _pre_layer(h, w, b, a):
    h4 = h.reshape(B * NB, BN, D)
    hw_sc, al_sc = pl.pallas_call(
        _pre_body,
        grid=(B * NB,),
        in_specs=[
            pl.BlockSpec((1, BN, D), lambda i: (i, 0, 0)),
            pl.BlockSpec((D, D), lambda i: (0, 0)),
            pl.BlockSpec((D,), lambda i: (0,)),
            pl.BlockSpec((D, 8), lambda i: (0, 0)),
        ],
        out_specs=[
            pl.BlockSpec((1, 2, BN, 64), lambda i: (i // NB, 0, i % NB, 0)),
            pl.BlockSpec((1, 2, BN, 4), lambda i: (i // NB, 0, i % NB, 0)),
        ],
        out_shape=[
            jax.ShapeDtypeStruct((B, 2, NP, 64), jnp.float32),
            jax.ShapeDtypeStruct((B, 2, NP, 4), jnp.float32),
        ],
    )(h4, w, b, a)
    return hw_sc, al_sc


def _post_body(h_ref, u0_ref, u1_ref, den_ref, s_ref, m_ref, o_ref):
    h = h_ref[0]
    unn = jnp.concatenate([u0_ref[0, 0], u1_ref[0, 0]], axis=-1)
    rep = _dotT(den_ref[0], s_ref[...])
    agg = unn / (rep + 1e-9)
    out = jnp.where(agg > 0.0, agg, jnp.exp(agg) - 1.0) + h
    o_ref[0] = out * m_ref[0, 0][:, None]


def _post_layer(h, unn_sc, den, sel, mask):
    h4 = h.reshape(B * NB, BN, D)
    den4 = den.reshape(B * NB, BN, H)
    m4 = mask.reshape(B * NB, 1, BN)
    out = pl.pallas_call(
        _post_body,
        grid=(B * NB,),
        in_specs=[
            pl.BlockSpec((1, BN, D), lambda i: (i, 0, 0)),
            pl.BlockSpec((1, 1, BN, 64), lambda i: (i // NB, 0, i % NB, 0)),
            pl.BlockSpec((1, 1, BN, 64), lambda i: (i // NB, 1, i % NB, 0)),
            pl.BlockSpec((1, BN, H), lambda i: (i, 0, 0)),
            pl.BlockSpec((H, D), lambda i: (0, 0)),
            pl.BlockSpec((1, 1, BN), lambda i: (i, 0, 0)),
        ],
        out_specs=pl.BlockSpec((1, BN, D), lambda i: (i, 0, 0)),
        out_shape=jax.ShapeDtypeStruct((B * NB, BN, D), jnp.float32),
    )(h4, unn_sc, unn_sc, den4, sel, m4)
    return out.reshape(B, N, D)


def _final_body(h_ref, w_ref, b_ref, m_ref, ne_ref, sums_ref):
    i = pl.program_id(1)
    h = h_ref[0, 0]
    ne = _dotT(h, w_ref[...]) + b_ref[...]
    ne_ref[0, 0] = ne
    mvec = m_ref[0, 0, 0]
    masked = ne * mvec[:, None]
    part = jnp.sum(masked, axis=0, keepdims=True)
    msum = jnp.sum(mvec)
    prow = jnp.concatenate([part, jnp.full((1, D), msum, jnp.float32)], axis=0)

    @pl.when(i == 0)
    def _():
        sums_ref[0] = prow

    @pl.when(i > 0)
    def _():
        sums_ref[0] = sums_ref[0] + prow


def _final(h, w, b, mask):
    h4 = h.reshape(B, NB, BN, D)
    m4 = mask.reshape(B, NB, 1, BN)
    ne, sums = pl.pallas_call(
        _final_body,
        grid=(B, NB),
        in_specs=[
            pl.BlockSpec((1, 1, BN, D), lambda bb, i: (bb, i, 0, 0)),
            pl.BlockSpec((D, D_OUT), lambda bb, i: (0, 0)),
            pl.BlockSpec((D_OUT,), lambda bb, i: (0,)),
            pl.BlockSpec((1, 1, 1, BN), lambda bb, i: (bb, i, 0, 0)),
        ],
        out_specs=[
            pl.BlockSpec((1, 1, BN, D_OUT), lambda bb, i: (bb, i, 0, 0)),
            pl.BlockSpec((1, 2, D_OUT), lambda bb, i: (bb, 0, 0)),
        ],
        out_shape=[
            jax.ShapeDtypeStruct((B, NB, BN, D_OUT), jnp.float32),
            jax.ShapeDtypeStruct((B, 2, D_OUT), jnp.float32),
        ],
    )(h4, w, b, m4)
    node_emb = ne.reshape(B, N, D_OUT)
    graph_emb = sums[:, 0, :] / jnp.clip(sums[:, 1, :], 1.0, None)
    return node_emb, graph_emb


def _sc_body(edge_hbm, al_hbm, hw_hbm, unn_out, den_out,
             alv, src_all, dst_all, denom_loc, rows, rows1,
             ex_buf, ex_buf1, zflat, idx_a, idx_b,
             unnorm_sh, denom_sh, sem_g, sem_g1, sem_s, sem_s1):
    c = lax.axis_index("c")
    tid = lax.axis_index("s")
    r0 = tid * STRIPE
    _F32Z = jnp.zeros((16,), jnp.float32)

    lanes = lax.iota(jnp.int32, 16)

    def fill_a(g, u):
        idx_a[pl.ds(g * 16, 16)] = lanes + g * 16
        return u
    lax.fori_loop(0, 8, fill_a, 0)

    def fill_b(g, u):
        idx_b[pl.ds(g * 16, 16)] = lanes + (128 + g * 16)
        return u
    lax.fori_loop(0, 2, fill_b, 0)

    def graph_body(b, carry):
        al_d = pltpu.async_copy(al_hbm.at[b, c], alv, sem_g)

        def z_rows(i, u):
            for q in range(4):
                rows[i, pl.ds(q * 16, 16)] = _F32Z
            return u
        lax.fori_loop(0, CH, z_rows, 0)

        def z_flat(i, u):
            for q in range(8):
                zflat[i, pl.ds(q * 16, 16)] = _F32Z
            return u
        lax.fori_loop(0, DR // 16, z_flat, 0)

        def z_den(i, u):
            for q in range(8):
                denom_loc[i, pl.ds(q * 16, 16)] = _F32Z
            return u
        lax.fori_loop(0, DR, z_den, 0)

        zds = []
        for k in range(4):
            zds.append(pltpu.async_copy(
                rows, unnorm_sh.at[pl.ds(r0 + k * 128, 128), :], sem_s))
        zds.append(pltpu.async_copy(
            rows.at[pl.ds(0, STRIPE - 512)],
            unnorm_sh.at[pl.ds(r0 + 512, STRIPE - 512), :], sem_s))
        zds.append(pltpu.async_copy(
            zflat, denom_sh.at[pl.ds(tid * (DR // 16), DR // 16), :], sem_s1))
        al_d.wait()
        for d in zds:
            d.wait()
        plsc.subcore_barrier()

        def compute_ex(j, exb):
            for g in range(CH // 16):
                s16 = src_all[j, pl.ds(g * 16, 16)]
                d16 = dst_all[j, pl.ds(g * 16, 16)]
                s4 = s16 * 4
                d4 = d16 * 4
                for h in range(2):
                    sval = plsc.load_gather(alv, [s4 + h])
                    dval = plsc.load_gather(alv, [d4 + (2 + h)])
                    e = sval + dval
                    e = jnp.where(e >= 0.0, e, e * jnp.float32(0.2))
                    exv = jnp.exp(e)
                    exb[pl.ds(g * 32 + h * 16, 16)] = exv
                    fi = d16 * 2 + h
                    plsc.addupdate_scatter(
                        denom_loc, [fi >> 7, fi & 127], exv)

        def scale_rows(buf, exb):
            for g in range(CH // 16):
                for h in range(2):
                    exv = exb[pl.ds(g * 32 + h * 16, 16)]
                    for l in range(16):
                        ee = g * 16 + l
                        scv = jnp.full((16,), exv[l], jnp.float32)
                        for q in range(2):
                            sl = pl.ds(h * 32 + q * 16, 16)
                            buf[ee, sl] = buf[ee, sl] * scv

        def drain_scatters():
            pltpu.make_async_copy(rows, unnorm_sh.at[dst_all.at[0]],
                                  sem_s).wait()
            pltpu.make_async_copy(rows1, unnorm_sh.at[dst_all.at[1]],
                                  sem_s1).wait()

        def pair(jj, u):
            j0 = jj * 2
            j1 = j0 + 1

            @pl.when(jj > 0)
            def _():
                drain_scatters()

            g0 = pltpu.async_copy(
                hw_hbm.at[b, c].at[src_all.at[j0]], rows, sem_g)
            g1 = pltpu.async_copy(
                hw_hbm.at[b, c].at[src_all.at[j1]], rows1, sem_g1)
            compute_ex(j0, ex_buf)
            compute_ex(j1, ex_buf1)
            g0.wait()
            scale_rows(rows, ex_buf)
            pltpu.async_copy(rows, unnorm_sh.at[dst_all.at[j0]],
                             sem_s, add=True)
            g1.wait()
            scale_rows(rows1, ex_buf1)
            pltpu.async_copy(rows1, unnorm_sh.at[dst_all.at[j1]],
                             sem_s1, add=True)
            return u

        def superblock(s, u):
            e0 = pltpu.async_copy(edge_hbm.at[b, 0, tid, s], src_all, sem_g)
            e1 = pltpu.async_copy(edge_hbm.at[b, 1, tid, s], dst_all, sem_g1)
            e0.wait()
            e1.wait()
            lax.fori_loop(0, SUP // 2, pair, 0)
            drain_scatters()
            return u

        lax.fori_loop(0, NSUP, superblock, 0)

        pltpu.sync_copy(denom_loc.at[pl.ds(0, 128)],
                        denom_sh.at[idx_a], add=True)
        pltpu.sync_copy(denom_loc.at[pl.ds(128, 32)],
                        denom_sh.at[idx_b], add=True)
        plsc.subcore_barrier()
        pltpu.sync_copy(unnorm_sh.at[pl.ds(r0, STRIPE), :],
                        unn_out.at[b, c, pl.ds(r0, STRIPE), :])
        pltpu.sync_copy(denom_sh.at[pl.ds(tid * (DR // 16), DR // 16), :],
                        den_out.at[b, c, tid])
        plsc.subcore_barrier()
        return carry

    lax.fori_loop(0, B, graph_body, 0)


@functools.partial(
    pl.kernel,
    mesh=plsc.VectorSubcoreMesh(core_axis_name="c", subcore_axis_name="s"),
    compiler_params=pltpu.CompilerParams(use_tc_tiling_on_sc=False,
                                         needs_layout_passes=False),
    out_type=[
        jax.ShapeDtypeStruct((B, 2, NP, 64), jnp.float32),
        jax.ShapeDtypeStruct((B, 2, 16, DR // 16, DC), jnp.float32),
    ],
    scratch_types=[
        pltpu.VMEM((ALW,), jnp.float32),
        pltpu.VMEM((SUP, CH), jnp.int32),
        pltpu.VMEM((SUP, CH), jnp.int32),
        pltpu.VMEM((DR, DC), jnp.float32),
        pltpu.VMEM((CH, 64), jnp.float32),
        pltpu.VMEM((CH, 64), jnp.float32),
        pltpu.VMEM((256,), jnp.float32),
        pltpu.VMEM((256,), jnp.float32),
        pltpu.VMEM((DR // 16, DC), jnp.float32),
        pltpu.VMEM((128,), jnp.int32),
        pltpu.VMEM((32,), jnp.int32),
        pltpu.VMEM_SHARED((NP, 64), jnp.float32),
        pltpu.VMEM_SHARED((DR, DC), jnp.float32),
        pltpu.SemaphoreType.DMA,
        pltpu.SemaphoreType.DMA,
        pltpu.SemaphoreType.DMA,
        pltpu.SemaphoreType.DMA,
    ],
)
def _sc_edge_kernel(edge_hbm, al_hbm, hw_hbm, unn_out, den_out,
                    alv, src_all, dst_all, denom_loc, rows, rows1,
                    ex_buf, ex_buf1, zflat, idx_a, idx_b,
                    unnorm_sh, denom_sh, sem_g, sem_g1, sem_s, sem_s1):
    _sc_body(edge_hbm, al_hbm, hw_hbm, unn_out, den_out,
             alv, src_all, dst_all, denom_loc, rows, rows1,
             ex_buf, ex_buf1, zflat, idx_a, idx_b,
             unnorm_sh, denom_sh, sem_g, sem_g1, sem_s, sem_s1)


def _layer(h, edge_sc, mask, w, b, a_sel, sel):
    hw_sc, al_sc = _pre_layer(h, w, b, a_sel)

    unn_sc, den_sc = _sc_edge_kernel(edge_sc, al_sc.reshape(B, 2, ALW), hw_sc)

    den = den_sc.reshape(B, 2, DR * DC // 2, 2)[:, :, :N, :]
    den = den.transpose(0, 2, 1, 3).reshape(B, N, H)

    return _post_layer(h, unn_sc, den, sel, mask)


def kernel(node_features, edge_index, node_types, node_mask, type_embed,
           in_W, in_b, gat_W, gat_b, a_src, a_dst, out_W, out_b):
    f32 = jnp.float32

    eyeh = jnp.repeat(jnp.eye(H, dtype=f32), HD, axis=0)
    a_s_mat = eyeh * a_src.reshape(L, 1, D).transpose(0, 2, 1)
    a_d_mat = eyeh * a_dst.reshape(L, 1, D).transpose(0, 2, 1)
    a_sel = jnp.concatenate([a_s_mat, a_d_mat], axis=-1)
    a_sel = a_sel[:, :, jnp.array([0, 1, 4, 5, 2, 3, 6, 7])]

    sel = jnp.repeat(jnp.eye(H, dtype=f32), HD, axis=1)

    ei = edge_index.astype(jnp.int32)
    ei_pad = jnp.concatenate(
        [ei, jnp.full((B, 2, EPAD - E), N, jnp.int32)], axis=2)
    edge_sc = ei_pad.reshape(B, 2, 16, NSUP, SUP, CH)

    h = _inproj(node_features, node_types, in_W, in_b, type_embed)
    for i in range(L):
        h = _layer(h, edge_sc, node_mask, gat_W[i], gat_b[i],
                   a_sel[i], sel)
    return _final(h, out_W, out_b, node_mask)
